# baseline (speedup 1.0000x reference)
import functools

import jax
import jax.numpy as jnp
from jax import lax
from jax.experimental import pallas as pl
from jax.experimental.pallas import tpu as pltpu
from jax.experimental.pallas import tpu_sc as plsc

N = 8192
NC = 2
NS = 16
NW = NC * NS
XPW = N // NW
XB = 4
L = 16
NYV = N // L

_mesh = plsc.VectorSubcoreMesh(core_axis_name="c", subcore_axis_name="s")


@functools.partial(
    pl.kernel,
    mesh=_mesh,
    out_type=[
        jax.ShapeDtypeStruct((NW, XPW * L), jnp.float32),
        jax.ShapeDtypeStruct((NW, N), jnp.float32),
    ],
    scratch_types=[
        pltpu.VMEM((3, N), jnp.float32),
        pltpu.VMEM((3, XPW), jnp.float32),
        pltpu.VMEM((N,), jnp.float32),
        pltpu.VMEM((XPW * L,), jnp.float32),
    ],
)
def _chamfer_sc(x_hbm, y_hbm, cx_out, cy_out, y_v, x_v, cy_v, cxa_v):
    cid = lax.axis_index("c")
    sid = lax.axis_index("s")
    wid = sid * NC + cid

    pltpu.sync_copy(y_hbm, y_v)
    pltpu.sync_copy(x_hbm.at[wid], x_v)

    inf16 = jnp.full((L,), jnp.inf, jnp.float32)

    def init_body(j, carry):
        cy_v[pl.ds(j * L, L)] = inf16
        return carry

    lax.fori_loop(0, NYV, init_body, 0)

    def outer(ib, cxsum):
        xv = [x_v[c, pl.ds(ib * L, L)] for c in range(3)]
        for sb in range(L // XB):
            xs = [[xv[c][sb * XB + k] for c in range(3)] for k in range(XB)]

            def inner(j, accs):
                y0 = y_v[0, pl.ds(j * L, L)]
                y1 = y_v[1, pl.ds(j * L, L)]
                y2 = y_v[2, pl.ds(j * L, L)]
                ds = []
                naccs = []
                for k in range(XB):
                    d = (jnp.abs(y0 - xs[k][0])
                         + jnp.abs(y1 - xs[k][1])
                         + jnp.abs(y2 - xs[k][2]))
                    ds.append(d)
                    naccs.append(jnp.minimum(accs[k], d))
                m = jnp.minimum(jnp.minimum(ds[0], ds[1]),
                                jnp.minimum(ds[2], ds[3]))
                cy_v[pl.ds(j * L, L)] = jnp.minimum(cy_v[pl.ds(j * L, L)], m)
                return tuple(naccs)

            accs = lax.fori_loop(0, NYV, inner,
                                 tuple(inf16 for _ in range(XB)))
            for k in range(XB):
                p = ib * L + sb * XB + k
                cxa_v[pl.ds(p * L, L)] = accs[k]
        return cxsum

    lax.fori_loop(0, XPW // L, outer, jnp.float32(0.0))

    pltpu.sync_copy(cxa_v, cx_out.at[wid])
    pltpu.sync_copy(cy_v, cy_out.at[wid])


def kernel(pc1, est_flow, pc2):
    x = (pc1 + est_flow)[0]
    y = pc2[0]
    xr = x.reshape(NW, XPW, 3).transpose(0, 2, 1)
    yr = y.T
    cx_vecs, cy_part = _chamfer_sc(xr, yr)
    cham_x_mean = jnp.mean(jnp.min(cx_vecs.reshape(N, L), axis=1))
    cham_y_mean = jnp.mean(jnp.min(cy_part, axis=0))
    loss = cham_x_mean + cham_y_mean
    return (loss, jnp.zeros((1, N), jnp.float32))


# speedup vs baseline: 1.7763x; 1.7763x over previous
import functools

import jax
import jax.numpy as jnp
from jax import lax
from jax.experimental import pallas as pl
from jax.experimental.pallas import tpu as pltpu
from jax.experimental.pallas import tpu_sc as plsc

N = 8192
NC = 2
NS = 16
NW = NC * NS
XPW = N // NW
XB = 4
L = 32
W = 16
NYV = N // L
NP = N // 2

_mesh = plsc.VectorSubcoreMesh(core_axis_name="c", subcore_axis_name="s")


@functools.partial(
    pl.kernel,
    mesh=_mesh,
    compiler_params=pltpu.CompilerParams(needs_layout_passes=False),
    out_type=[
        jax.ShapeDtypeStruct((NW, XPW * W), jnp.int32),
        jax.ShapeDtypeStruct((NW, NP), jnp.int32),
    ],
    scratch_types=[
        pltpu.VMEM((3, NP), jnp.int32),
        pltpu.VMEM((3, XPW), jnp.int32),
        pltpu.VMEM((NP,), jnp.int32),
        pltpu.VMEM((XPW * W,), jnp.int32),
    ],
)
def _chamfer_sc(x_hbm, y_hbm, cx_out, cy_out, y_v, x_v, cy_v, cxa_v):
    cid = lax.axis_index("c")
    sid = lax.axis_index("s")
    wid = sid * NC + cid

    pltpu.sync_copy(y_hbm, y_v)
    pltpu.sync_copy(x_hbm.at[wid], x_v)

    infv = jnp.full((L,), jnp.inf, jnp.bfloat16)
    infw = plsc.bitcast(infv, jnp.int32)

    def init_body(j, carry):
        cy_v[pl.ds(j * W, W)] = infw
        return carry

    lax.fori_loop(0, NYV, init_body, 0)

    def outer(ib, carry):
        xv = [x_v[c, pl.ds(ib * 16, 16)] for c in range(3)]
        for sb in range(16 // XB):
            xs = [[plsc.bitcast(jnp.full((W,), xv[c][sb * XB + k],
                                         jnp.int32), jnp.bfloat16)
                   for c in range(3)] for k in range(XB)]

            def inner(j, accs):
                y0 = plsc.bitcast(y_v[0, pl.ds(j * W, W)], jnp.bfloat16)
                y1 = plsc.bitcast(y_v[1, pl.ds(j * W, W)], jnp.bfloat16)
                y2 = plsc.bitcast(y_v[2, pl.ds(j * W, W)], jnp.bfloat16)
                ds = []
                naccs = []
                for k in range(XB):
                    d = (jnp.abs(y0 - xs[k][0])
                         + jnp.abs(y1 - xs[k][1])
                         + jnp.abs(y2 - xs[k][2]))
                    ds.append(d)
                    naccs.append(jnp.minimum(accs[k], d))
                m = jnp.minimum(jnp.minimum(ds[0], ds[1]),
                                jnp.minimum(ds[2], ds[3]))
                cyv = plsc.bitcast(cy_v[pl.ds(j * W, W)], jnp.bfloat16)
                cy_v[pl.ds(j * W, W)] = plsc.bitcast(
                    jnp.minimum(cyv, m), jnp.int32)
                return tuple(naccs)

            accs = lax.fori_loop(0, NYV, inner,
                                 tuple(infv for _ in range(XB)))
            for k in range(XB):
                p = ib * 16 + sb * XB + k
                cxa_v[pl.ds(p * W, W)] = plsc.bitcast(accs[k], jnp.int32)
        return carry

    lax.fori_loop(0, XPW // 16, outer, 0)

    pltpu.sync_copy(cxa_v, cx_out.at[wid])
    pltpu.sync_copy(cy_v, cy_out.at[wid])


def _unpack_bf16(a_i32):
    b = lax.bitcast_convert_type(a_i32, jnp.bfloat16)
    return b.astype(jnp.float32)


def kernel(pc1, est_flow, pc2):
    x = (pc1 + est_flow)[0]
    y = pc2[0]
    xu = lax.bitcast_convert_type(x.astype(jnp.bfloat16),
                                  jnp.uint16).astype(jnp.uint32)
    xdup = lax.bitcast_convert_type(xu | (xu << 16), jnp.int32)
    xr = xdup.reshape(NW, XPW, 3).transpose(0, 2, 1)
    yb = y.T.astype(jnp.bfloat16).reshape(3, NP, 2)
    yr = lax.bitcast_convert_type(yb, jnp.int32)
    cx_vecs, cy_part = _chamfer_sc(xr, yr)
    cx = _unpack_bf16(cx_vecs).reshape(N, L)
    cham_x_mean = jnp.mean(jnp.min(cx, axis=1))
    cy = _unpack_bf16(cy_part).reshape(NW, N)
    cham_y_mean = jnp.mean(jnp.min(cy, axis=0))
    loss = cham_x_mean + cham_y_mean
    return (loss, jnp.zeros((1, N), jnp.float32))


# speedup vs baseline: 2.6236x; 1.4770x over previous
import functools

import jax
import jax.numpy as jnp
from jax import lax
from jax.experimental import pallas as pl
from jax.experimental.pallas import tpu as pltpu
from jax.experimental.pallas import tpu_sc as plsc

N = 8192
NC = 2
NS = 16
NW = NC * NS
NTC = 4096
NSC = N - NTC
XPW = NSC // NW
XB = 4
L = 32
W = 16
NYV = N // L
NP = N // 2

BY = 256
NYB = N // BY

_mesh = plsc.VectorSubcoreMesh(core_axis_name="c", subcore_axis_name="s")


@functools.partial(
    pl.kernel,
    mesh=_mesh,
    compiler_params=pltpu.CompilerParams(needs_layout_passes=False),
    out_type=[
        jax.ShapeDtypeStruct((NW, XPW * W), jnp.int32),
        jax.ShapeDtypeStruct((NW, NP), jnp.int32),
    ],
    scratch_types=[
        pltpu.VMEM((3, NP), jnp.int32),
        pltpu.VMEM((3, XPW), jnp.int32),
        pltpu.VMEM((NP,), jnp.int32),
        pltpu.VMEM((XPW * W,), jnp.int32),
    ],
)
def _chamfer_sc(x_hbm, y_hbm, cx_out, cy_out, y_v, x_v, cy_v, cxa_v):
    cid = lax.axis_index("c")
    sid = lax.axis_index("s")
    wid = sid * NC + cid

    pltpu.sync_copy(y_hbm, y_v)
    pltpu.sync_copy(x_hbm.at[wid], x_v)

    infv = jnp.full((L,), jnp.inf, jnp.bfloat16)
    infw = plsc.bitcast(infv, jnp.int32)

    def init_body(j, carry):
        cy_v[pl.ds(j * W, W)] = infw
        return carry

    lax.fori_loop(0, NYV, init_body, 0)

    def outer(ib, carry):
        xv = [x_v[c, pl.ds(ib * 16, 16)] for c in range(3)]
        for sb in range(16 // XB):
            xs = [[plsc.bitcast(jnp.full((W,), xv[c][sb * XB + k],
                                         jnp.int32), jnp.bfloat16)
                   for c in range(3)] for k in range(XB)]

            def inner(j, accs):
                y0 = plsc.bitcast(y_v[0, pl.ds(j * W, W)], jnp.bfloat16)
                y1 = plsc.bitcast(y_v[1, pl.ds(j * W, W)], jnp.bfloat16)
                y2 = plsc.bitcast(y_v[2, pl.ds(j * W, W)], jnp.bfloat16)
                ds = []
                naccs = []
                for k in range(XB):
                    d = (jnp.abs(y0 - xs[k][0])
                         + jnp.abs(y1 - xs[k][1])
                         + jnp.abs(y2 - xs[k][2]))
                    ds.append(d)
                    naccs.append(jnp.minimum(accs[k], d))
                m = jnp.minimum(jnp.minimum(ds[0], ds[1]),
                                jnp.minimum(ds[2], ds[3]))
                cyv = plsc.bitcast(cy_v[pl.ds(j * W, W)], jnp.bfloat16)
                cy_v[pl.ds(j * W, W)] = plsc.bitcast(
                    jnp.minimum(cyv, m), jnp.int32)
                return tuple(naccs)

            accs = lax.fori_loop(0, NYV, inner,
                                 tuple(infv for _ in range(XB)))
            for k in range(XB):
                p = ib * 16 + sb * XB + k
                cxa_v[pl.ds(p * W, W)] = plsc.bitcast(accs[k], jnp.int32)
        return carry

    lax.fori_loop(0, XPW // 16, outer, 0)

    pltpu.sync_copy(cxa_v, cx_out.at[wid])
    pltpu.sync_copy(cy_v, cy_out.at[wid])


def _tc_body(x_ref, y_ref, cx_ref, cy_ref):
    j = pl.program_id(0)
    x0 = x_ref[0, :][:, None]
    x1 = x_ref[1, :][:, None]
    x2 = x_ref[2, :][:, None]
    y0 = y_ref[0, :][None, :]
    y1 = y_ref[1, :][None, :]
    y2 = y_ref[2, :][None, :]
    d = (jnp.abs(x0 - y0) + jnp.abs(x1 - y1) + jnp.abs(x2 - y2))
    cy_ref[...] = jnp.min(d, axis=0)
    m = jnp.min(d, axis=1)

    @pl.when(j == 0)
    def _():
        cx_ref[...] = m

    @pl.when(j > 0)
    def _():
        cx_ref[...] = jnp.minimum(cx_ref[...], m)


_chamfer_tc = pl.pallas_call(
    _tc_body,
    grid=(NYB,),
    in_specs=[
        pl.BlockSpec((3, NTC), lambda j: (0, 0)),
        pl.BlockSpec((3, BY), lambda j: (0, j)),
    ],
    out_specs=[
        pl.BlockSpec((NTC,), lambda j: (0,)),
        pl.BlockSpec((BY,), lambda j: (j,)),
    ],
    out_shape=[
        jax.ShapeDtypeStruct((NTC,), jnp.float32),
        jax.ShapeDtypeStruct((N,), jnp.float32),
    ],
)


def _unpack_bf16(a_i32):
    b = lax.bitcast_convert_type(a_i32, jnp.bfloat16)
    return b.astype(jnp.float32)


def kernel(pc1, est_flow, pc2):
    x = (pc1 + est_flow)[0]
    y = pc2[0]
    yt = y.T

    xsc = x[NTC:]
    xu = lax.bitcast_convert_type(xsc.astype(jnp.bfloat16),
                                  jnp.uint16).astype(jnp.uint32)
    xdup = lax.bitcast_convert_type(xu | (xu << 16), jnp.int32)
    xr = xdup.reshape(NW, XPW, 3).transpose(0, 2, 1)
    yb = yt.astype(jnp.bfloat16).reshape(3, NP, 2)
    yr = lax.bitcast_convert_type(yb, jnp.int32)

    cx_vecs, cy_sc = _chamfer_sc(xr, yr)

    cx_tc, cy_tc = _chamfer_tc(x[:NTC].T, yt)

    cx_sc = jnp.min(_unpack_bf16(cx_vecs).reshape(NSC, L), axis=1)
    cham_x_sum = jnp.sum(cx_sc) + jnp.sum(cx_tc)
    cy = jnp.minimum(jnp.min(_unpack_bf16(cy_sc).reshape(NW, N), axis=0),
                     cy_tc)
    loss = cham_x_sum / N + jnp.mean(cy)
    return (loss, jnp.zeros((1, N), jnp.float32))
